# baseline (speedup 1.0000x reference)
import functools

import jax
import jax.numpy as jnp
from jax.experimental import pallas as pl
from jax.experimental.pallas import tpu as pltpu

_G = 16
_EPS = 1e-08
_OFFS = (-3, -2, -1, 1, 2, 3)
_CPAD = 128
_HALO = 8


def _csum3(a):
    return a[0:_G] + a[_G:2 * _G] + a[2 * _G:3 * _G]


def _rep3(a):
    return jnp.concatenate([a, a, a], axis=0)


def _fused_kernel(wgt_ref, vt_ref, xp_ref, mlpq_ref, mlpk_ref, weq_ref,
                  wek_ref, q_ref, k_ref, vgs_ref, *, nb, n_nodes, nblk):
    j = pl.program_id(1)
    n2 = nblk * nb

    @pl.when(j == 0)
    def _():
        vgs_ref[:, :_CPAD] = jnp.zeros((3 * _G, _CPAD), jnp.float32)
        vgs_ref[:, _CPAD + n2:] = jnp.zeros((3 * _G, _CPAD), jnp.float32)

    @pl.when(j < nblk)
    def _():
        dst = pl.multiple_of(_CPAD + j * nb, 128)
        for c in range(3):
            r = jax.lax.dot_general(
                wgt_ref[...], vt_ref[0, c], (((1,), (1,)), ((), ())),
                preferred_element_type=jnp.float32)
            vgs_ref[c * _G:(c + 1) * _G, pl.ds(dst, nb)] = r

    @pl.when(j > 0)
    def _():
        s = (j - 1) * nb
        nw = nb + 2 * _HALO
        c0 = _CPAD - _HALO

        win = vgs_ref[:, pl.ds(pl.multiple_of(s, 128), nb + 2 * _CPAD)]
        xw = xp_ref[0, pl.ds(pl.multiple_of(s, 8), nw), :]

        vg_w = win[:, c0:c0 + nw]
        node = jax.lax.broadcasted_iota(jnp.int32, (1, nw), 1) + (s - _HALO)
        n_ok = (node >= 0) & (node < n_nodes)
        u = jnp.zeros((3 * _G, nw), jnp.float32)
        es = []
        for o in _OFFS:
            vg_j = win[:, c0 + o:c0 + o + nw]
            bb = vg_j - vg_w
            rn = 1.0 / jnp.maximum(jnp.sqrt(_csum3(bb * bb)), _EPS)
            e = bb * _rep3(rn)
            es.append(e)
            mask = n_ok & (node + o >= 0) & (node + o < n_nodes)
            u = u + jnp.where(mask, e, 0.0)

        u_s = u[:, _HALO:_HALO + nb]
        rnu = 1.0 / jnp.maximum(jnp.sqrt(_csum3(u_s * u_s)), _EPS)

        q_ref[0] = jnp.zeros((nb, xp_ref.shape[2]), jnp.float32)
        k_ref[0] = jnp.zeros((nb, xp_ref.shape[2]), jnp.float32)
        ones = jnp.ones((1, nb), jnp.float32)

        for oi, o in enumerate(_OFFS):
            e = es[oi][:, _HALO:_HALO + nb]
            u_j = u[:, _HALO + o:_HALO + o + nb]
            d_ue = _csum3(u_s * e)
            ang = jnp.clip(d_ue * rnu, -1.0, 1.0)
            d_uje = _csum3(u_j * e)
            ui_p = u_s - _rep3(d_ue) * e
            uj_p = u_j - _rep3(d_uje) * e
            dotp = _csum3(ui_p * uj_p)
            npi = jnp.maximum(jnp.sqrt(_csum3(ui_p * ui_p)), _EPS)
            npj = jnp.maximum(jnp.sqrt(_csum3(uj_p * uj_p)), _EPS)
            dih = jnp.clip(dotp / jnp.maximum(npi * npj, _EPS), -1.0, 1.0)

            def mlp(p_ref):
                raw = jnp.zeros((_G, nb), jnp.float32)
                for m in range(16):
                    th = ang * p_ref[0, m] + dih * p_ref[1, m] + p_ref[2, m]
                    raw = raw + p_ref[3, m] * (th * (1.0 + jnp.tanh(th)))
                return raw

            x_j = 0.5 * xw[_HALO + o:_HALO + o + nb, :]
            for p_ref, w_ref, o_ref in (
                    (mlpq_ref, weq_ref, q_ref),
                    (mlpk_ref, wek_ref, k_ref)):
                raw = jnp.concatenate([mlp(p_ref), ones], axis=0)
                logits = jax.lax.dot_general(
                    raw, w_ref[...], (((0,), (0,)), ((), ())),
                    preferred_element_type=jnp.float32)
                gate = 1.0 + jnp.tanh(logits)
                o_ref[0] = o_ref[0] + gate * x_j


def kernel(x_scalar, vec, W_geom, Wq1, bq1, Wq2, bq2, Wk1, bk1, Wk2, bk2,
           Weq, beq, Wek, bek):
    B, N, H = x_scalar.shape
    G = W_geom.shape[1]
    nb = 3456 if N >= 3456 else ((N + 7) // 8) * 8
    nblk = -(-N // nb)
    n2 = nblk * nb

    vec_t = jnp.pad(vec.transpose(0, 2, 1, 3),
                    ((0, 0), (0, 0), (0, n2 - N), (0, 0)))
    wgt = W_geom.T
    xp = jnp.pad(x_scalar, ((0, 0), (_HALO, n2 - N + 2 * _HALO), (0, 0)))

    mlpq = jnp.stack([0.5 * Wq1[0], 0.5 * Wq1[1], 0.5 * bq1, Wq2[:, 0]],
                     axis=0)
    mlpk = jnp.stack([0.5 * Wk1[0], 0.5 * Wk1[1], 0.5 * bk1, Wk2[:, 0]],
                     axis=0)
    beq_eff = 0.5 * (beq + bq2[0] * jnp.sum(Weq, axis=0)).reshape(1, H)
    bek_eff = 0.5 * (bek + bk2[0] * jnp.sum(Wek, axis=0)).reshape(1, H)
    weq_h = jnp.concatenate([0.5 * Weq, beq_eff], axis=0)
    wek_h = jnp.concatenate([0.5 * Wek, bek_eff], axis=0)

    last = nblk - 1
    q2, k2 = pl.pallas_call(
        functools.partial(_fused_kernel, nb=nb, n_nodes=N, nblk=nblk),
        grid=(B, nblk + 1),
        in_specs=[
            pl.BlockSpec((G, H), lambda b, j: (0, 0)),
            pl.BlockSpec((1, 3, nb, H),
                         lambda b, j: (b, 0, jnp.minimum(j, last), 0)),
            pl.BlockSpec((1, n2 + 2 * _HALO, H), lambda b, j: (b, 0, 0)),
            pl.BlockSpec(memory_space=pltpu.SMEM),
            pl.BlockSpec(memory_space=pltpu.SMEM),
            pl.BlockSpec((G + 1, H), lambda b, j: (0, 0)),
            pl.BlockSpec((G + 1, H), lambda b, j: (0, 0)),
        ],
        out_specs=[
            pl.BlockSpec((1, nb, H), lambda b, j: (b, jnp.maximum(j - 1, 0), 0)),
            pl.BlockSpec((1, nb, H), lambda b, j: (b, jnp.maximum(j - 1, 0), 0)),
        ],
        out_shape=[
            jax.ShapeDtypeStruct((B, N, H), jnp.float32),
            jax.ShapeDtypeStruct((B, N, H), jnp.float32),
        ],
        scratch_shapes=[pltpu.VMEM((3 * G, n2 + 2 * _CPAD), jnp.float32)],
    )(wgt, vec_t, xp, mlpq, mlpk, weq_h, wek_h)

    return (q2, k2)


# speedup vs baseline: 1.0282x; 1.0282x over previous
import functools

import jax
import jax.numpy as jnp
from jax.experimental import pallas as pl
from jax.experimental.pallas import tpu as pltpu

_G = 16
_EPS = 1e-08
_OFFS = (-3, -2, -1, 1, 2, 3)
_CPAD = 128
_HALO = 8


def _csum3(a):
    return a[0:_G] + a[_G:2 * _G] + a[2 * _G:3 * _G]


def _rep3(a):
    return jnp.concatenate([a, a, a], axis=0)


def _fused_kernel(wgt_ref, vt_ref, xp_ref, mlpq_ref, mlpk_ref, weq_ref,
                  wek_ref, q_ref, k_ref, vgs_ref, *, nb, n_nodes, nblk):
    j = pl.program_id(1)
    n2 = nblk * nb

    @pl.when(j == 0)
    def _():
        vgs_ref[:, :_CPAD] = jnp.zeros((3 * _G, _CPAD), jnp.float32)
        vgs_ref[:, _CPAD + n2:] = jnp.zeros((3 * _G, _CPAD), jnp.float32)

    @pl.when(j < nblk)
    def _():
        dst = pl.multiple_of(_CPAD + j * nb, 128)
        for c in range(3):
            r = jax.lax.dot_general(
                wgt_ref[...], vt_ref[0, c], (((1,), (1,)), ((), ())),
                preferred_element_type=jnp.float32)
            vgs_ref[c * _G:(c + 1) * _G, pl.ds(dst, nb)] = r

    @pl.when(j > 0)
    def _():
        s = (j - 1) * nb
        nw = nb + 2 * _HALO
        c0 = _CPAD - _HALO

        win = vgs_ref[:, pl.ds(pl.multiple_of(s, 128), nb + 2 * _CPAD)]
        xw = xp_ref[0, pl.ds(pl.multiple_of(s, 8), nw), :]

        vg_w = win[:, c0:c0 + nw]
        node = jax.lax.broadcasted_iota(jnp.int32, (1, nw), 1) + (s - _HALO)
        n_ok = (node >= 0) & (node < n_nodes)
        u = jnp.zeros((3 * _G, nw), jnp.float32)
        es = []
        for o in _OFFS:
            vg_j = win[:, c0 + o:c0 + o + nw]
            bb = vg_j - vg_w
            rn = 1.0 / jnp.maximum(jnp.sqrt(_csum3(bb * bb)), _EPS)
            e = bb * _rep3(rn)
            es.append(e)
            mask = n_ok & (node + o >= 0) & (node + o < n_nodes)
            u = u + jnp.where(mask, e, 0.0)

        u_s = u[:, _HALO:_HALO + nb]
        rnu = 1.0 / jnp.maximum(jnp.sqrt(_csum3(u_s * u_s)), _EPS)

        q_ref[0] = jnp.zeros((nb, xp_ref.shape[2]), jnp.float32)
        k_ref[0] = jnp.zeros((nb, xp_ref.shape[2]), jnp.float32)
        ones = jnp.ones((1, nb), jnp.float32)

        def mlp_gate(ang, dih, o):
            def mlp(p_ref):
                raw = jnp.zeros((_G, nb), jnp.float32)
                for m in range(16):
                    th = ang * p_ref[0, m] + dih * p_ref[1, m] + p_ref[2, m]
                    raw = raw + p_ref[3, m] * (th * (1.0 + jnp.tanh(th)))
                return raw

            x_j = 0.5 * xw[_HALO + o:_HALO + o + nb, :]
            for p_ref, w_ref, o_ref in (
                    (mlpq_ref, weq_ref, q_ref),
                    (mlpk_ref, wek_ref, k_ref)):
                raw = jnp.concatenate([mlp(p_ref), ones], axis=0)
                logits = jax.lax.dot_general(
                    raw, w_ref[...], (((0,), (0,)), ((), ())),
                    preferred_element_type=jnp.float32)
                gate = 1.0 + jnp.tanh(logits)
                o_ref[0] = o_ref[0] + gate * x_j

        c4 = _HALO - 4
        ne = nb + 8
        for oi, o in ((3, 1), (4, 2), (5, 3)):
            e_w = es[oi][:, c4:c4 + ne]
            u_w = u[:, c4:c4 + ne]
            u_jw = u[:, c4 + o:c4 + o + ne]
            d1 = _csum3(u_w * e_w)
            d2 = _csum3(u_jw * e_w)
            ui_p = u_w - _rep3(d1) * e_w
            uj_p = u_jw - _rep3(d2) * e_w
            dotp = _csum3(ui_p * uj_p)
            npi = jnp.maximum(jnp.sqrt(_csum3(ui_p * ui_p)), _EPS)
            npj = jnp.maximum(jnp.sqrt(_csum3(uj_p * uj_p)), _EPS)
            dih_w = jnp.clip(dotp / jnp.maximum(npi * npj, _EPS),
                             -1.0, 1.0)
            mlp_gate(jnp.clip(d1[:, 4:4 + nb] * rnu, -1.0, 1.0),
                     dih_w[:, 4:4 + nb], o)
            mlp_gate(jnp.clip(-d2[:, 4 - o:4 - o + nb] * rnu, -1.0, 1.0),
                     dih_w[:, 4 - o:4 - o + nb], -o)


def kernel(x_scalar, vec, W_geom, Wq1, bq1, Wq2, bq2, Wk1, bk1, Wk2, bk2,
           Weq, beq, Wek, bek):
    B, N, H = x_scalar.shape
    G = W_geom.shape[1]
    nb = 2560 if N >= 2560 else ((N + 7) // 8) * 8
    nblk = -(-N // nb)
    n2 = nblk * nb

    vec_t = jnp.pad(vec.transpose(0, 2, 1, 3),
                    ((0, 0), (0, 0), (0, n2 - N), (0, 0)))
    wgt = W_geom.T
    xp = jnp.pad(x_scalar, ((0, 0), (_HALO, n2 - N + 2 * _HALO), (0, 0)))

    mlpq = jnp.stack([0.5 * Wq1[0], 0.5 * Wq1[1], 0.5 * bq1, Wq2[:, 0]],
                     axis=0)
    mlpk = jnp.stack([0.5 * Wk1[0], 0.5 * Wk1[1], 0.5 * bk1, Wk2[:, 0]],
                     axis=0)
    beq_eff = 0.5 * (beq + bq2[0] * jnp.sum(Weq, axis=0)).reshape(1, H)
    bek_eff = 0.5 * (bek + bk2[0] * jnp.sum(Wek, axis=0)).reshape(1, H)
    weq_h = jnp.concatenate([0.5 * Weq, beq_eff], axis=0)
    wek_h = jnp.concatenate([0.5 * Wek, bek_eff], axis=0)

    last = nblk - 1
    q2, k2 = pl.pallas_call(
        functools.partial(_fused_kernel, nb=nb, n_nodes=N, nblk=nblk),
        grid=(B, nblk + 1),
        in_specs=[
            pl.BlockSpec((G, H), lambda b, j: (0, 0)),
            pl.BlockSpec((1, 3, nb, H),
                         lambda b, j: (b, 0, jnp.minimum(j, last), 0)),
            pl.BlockSpec((1, n2 + 2 * _HALO, H), lambda b, j: (b, 0, 0)),
            pl.BlockSpec(memory_space=pltpu.SMEM),
            pl.BlockSpec(memory_space=pltpu.SMEM),
            pl.BlockSpec((G + 1, H), lambda b, j: (0, 0)),
            pl.BlockSpec((G + 1, H), lambda b, j: (0, 0)),
        ],
        out_specs=[
            pl.BlockSpec((1, nb, H), lambda b, j: (b, jnp.maximum(j - 1, 0), 0)),
            pl.BlockSpec((1, nb, H), lambda b, j: (b, jnp.maximum(j - 1, 0), 0)),
        ],
        out_shape=[
            jax.ShapeDtypeStruct((B, N, H), jnp.float32),
            jax.ShapeDtypeStruct((B, N, H), jnp.float32),
        ],
        scratch_shapes=[pltpu.VMEM((3 * G, n2 + 2 * _CPAD), jnp.float32)],
    )(wgt, vec_t, xp, mlpq, mlpk, weq_h, wek_h)

    return (q2, k2)


# speedup vs baseline: 1.0534x; 1.0245x over previous
import functools

import jax
import jax.numpy as jnp
from jax.experimental import pallas as pl
from jax.experimental.pallas import tpu as pltpu

_G = 16
_EPS = 1e-08
_OFFS = (-3, -2, -1, 1, 2, 3)
_CPAD = 128
_HALO = 8


def _csum3(a):
    return a[0:_G] + a[_G:2 * _G] + a[2 * _G:3 * _G]


def _rep3(a):
    return jnp.concatenate([a, a, a], axis=0)


def _fused_kernel(wgt_ref, vt_ref, xp_ref, mlpq_ref, mlpk_ref, weq_ref,
                  wek_ref, q_ref, k_ref, vgs_ref, *, nb, n_nodes, nblk):
    j = pl.program_id(1)
    n2 = nblk * nb

    @pl.when(j == 0)
    def _():
        vgs_ref[:, :_CPAD] = jnp.zeros((3 * _G, _CPAD), jnp.float32)
        vgs_ref[:, _CPAD + n2:] = jnp.zeros((3 * _G, _CPAD), jnp.float32)

    @pl.when(j < nblk)
    def _():
        dst = pl.multiple_of(_CPAD + j * nb, 128)
        for c in range(3):
            r = jax.lax.dot_general(
                wgt_ref[...], vt_ref[0, c], (((1,), (1,)), ((), ())),
                preferred_element_type=jnp.float32)
            vgs_ref[c * _G:(c + 1) * _G, pl.ds(dst, nb)] = r

    @pl.when(j > 0)
    def _():
        s = (j - 1) * nb
        nw = nb + 2 * _HALO
        c0 = _CPAD - _HALO

        win = vgs_ref[:, pl.ds(pl.multiple_of(s, 128), nb + 2 * _CPAD)]
        xw = xp_ref[0, pl.ds(pl.multiple_of(s, 8), nw), :]

        nwx = nw + 4
        cx = c0 - 4
        vg_wx = win[:, cx:cx + nwx]
        node = jax.lax.broadcasted_iota(jnp.int32, (1, nwx), 1) + (s - 12)
        n_ok = (node >= 0) & (node < n_nodes)
        es, mes = [], []
        for o in (1, 2, 3):
            vg_j = win[:, cx + o:cx + o + nwx]
            bb = vg_j - vg_wx
            rn = 1.0 / jnp.maximum(jnp.sqrt(_csum3(bb * bb)), _EPS)
            e = bb * _rep3(rn)
            es.append(e)
            mes.append(jnp.where(n_ok & (node + o < n_nodes), e, 0.0))
        u = (mes[0][:, 4:4 + nw] + mes[1][:, 4:4 + nw]
             + mes[2][:, 4:4 + nw] - mes[0][:, 3:3 + nw]
             - mes[1][:, 2:2 + nw] - mes[2][:, 1:1 + nw])

        u_s = u[:, _HALO:_HALO + nb]
        rnu = 1.0 / jnp.maximum(jnp.sqrt(_csum3(u_s * u_s)), _EPS)

        q_ref[0] = jnp.zeros((nb, xp_ref.shape[2]), jnp.float32)
        k_ref[0] = jnp.zeros((nb, xp_ref.shape[2]), jnp.float32)
        ones = jnp.ones((1, nb), jnp.float32)

        def mlp_gate(ang, dih, o):
            def mlp(p_ref):
                raw = jnp.zeros((_G, nb), jnp.float32)
                for m in range(16):
                    th = ang * p_ref[0, m] + dih * p_ref[1, m] + p_ref[2, m]
                    raw = raw + p_ref[3, m] * (th * (1.0 + jnp.tanh(th)))
                return raw

            x_j = xw[_HALO + o:_HALO + o + nb, :]
            for p_ref, w_ref, o_ref in (
                    (mlpq_ref, weq_ref, q_ref),
                    (mlpk_ref, wek_ref, k_ref)):
                raw = jnp.concatenate([mlp(p_ref), ones], axis=0)
                logits = jax.lax.dot_general(
                    raw, w_ref[...], (((0,), (0,)), ((), ())),
                    preferred_element_type=jnp.float32)
                gate = 1.0 + jnp.tanh(logits)
                o_ref[0] = o_ref[0] + gate * x_j

        c4 = _HALO - 4
        ne = nb + 8
        for oi, o in ((0, 1), (1, 2), (2, 3)):
            e_w = es[oi][:, 8:8 + ne]
            u_w = u[:, c4:c4 + ne]
            u_jw = u[:, c4 + o:c4 + o + ne]
            d1 = _csum3(u_w * e_w)
            d2 = _csum3(u_jw * e_w)
            ui_p = u_w - _rep3(d1) * e_w
            uj_p = u_jw - _rep3(d2) * e_w
            dotp = _csum3(ui_p * uj_p)
            npi = jnp.maximum(jnp.sqrt(_csum3(ui_p * ui_p)), _EPS)
            npj = jnp.maximum(jnp.sqrt(_csum3(uj_p * uj_p)), _EPS)
            dih_w = jnp.clip(dotp / jnp.maximum(npi * npj, _EPS),
                             -1.0, 1.0)
            mlp_gate(jnp.clip(d1[:, 4:4 + nb] * rnu, -1.0, 1.0),
                     dih_w[:, 4:4 + nb], o)
            mlp_gate(jnp.clip(-d2[:, 4 - o:4 - o + nb] * rnu, -1.0, 1.0),
                     dih_w[:, 4 - o:4 - o + nb], -o)


def kernel(x_scalar, vec, W_geom, Wq1, bq1, Wq2, bq2, Wk1, bk1, Wk2, bk2,
           Weq, beq, Wek, bek):
    B, N, H = x_scalar.shape
    G = W_geom.shape[1]
    nb = 2560 if N >= 2560 else ((N + 7) // 8) * 8
    nblk = -(-N // nb)
    n2 = nblk * nb

    vec_t = jnp.pad(vec.transpose(0, 2, 1, 3),
                    ((0, 0), (0, 0), (0, n2 - N), (0, 0)))
    wgt = W_geom.T
    xp = jnp.pad(0.5 * x_scalar, ((0, 0), (_HALO, n2 - N + 2 * _HALO), (0, 0)))

    mlpq = jnp.stack([0.5 * Wq1[0], 0.5 * Wq1[1], 0.5 * bq1, Wq2[:, 0]],
                     axis=0)
    mlpk = jnp.stack([0.5 * Wk1[0], 0.5 * Wk1[1], 0.5 * bk1, Wk2[:, 0]],
                     axis=0)
    beq_eff = 0.5 * (beq + bq2[0] * jnp.sum(Weq, axis=0)).reshape(1, H)
    bek_eff = 0.5 * (bek + bk2[0] * jnp.sum(Wek, axis=0)).reshape(1, H)
    weq_h = jnp.concatenate([0.5 * Weq, beq_eff], axis=0)
    wek_h = jnp.concatenate([0.5 * Wek, bek_eff], axis=0)

    last = nblk - 1
    q2, k2 = pl.pallas_call(
        functools.partial(_fused_kernel, nb=nb, n_nodes=N, nblk=nblk),
        grid=(B, nblk + 1),
        in_specs=[
            pl.BlockSpec((G, H), lambda b, j: (0, 0)),
            pl.BlockSpec((1, 3, nb, H),
                         lambda b, j: (b, 0, jnp.minimum(j, last), 0)),
            pl.BlockSpec((1, n2 + 2 * _HALO, H), lambda b, j: (b, 0, 0)),
            pl.BlockSpec(memory_space=pltpu.SMEM),
            pl.BlockSpec(memory_space=pltpu.SMEM),
            pl.BlockSpec((G + 1, H), lambda b, j: (0, 0)),
            pl.BlockSpec((G + 1, H), lambda b, j: (0, 0)),
        ],
        out_specs=[
            pl.BlockSpec((1, nb, H), lambda b, j: (b, jnp.maximum(j - 1, 0), 0)),
            pl.BlockSpec((1, nb, H), lambda b, j: (b, jnp.maximum(j - 1, 0), 0)),
        ],
        out_shape=[
            jax.ShapeDtypeStruct((B, N, H), jnp.float32),
            jax.ShapeDtypeStruct((B, N, H), jnp.float32),
        ],
        scratch_shapes=[pltpu.VMEM((3 * G, n2 + 2 * _CPAD), jnp.float32)],
    )(wgt, vec_t, xp, mlpq, mlpk, weq_h, wek_h)

    return (q2, k2)
